# baseline (speedup 1.0000x reference)
import functools

import jax
import jax.numpy as jnp
import numpy as np
from jax import lax
from jax.experimental import pallas as pl
from jax.experimental.pallas import tpu as pltpu
from jax.experimental.pallas import tpu_sc as plsc

_LV = np.array(
    [-1.0, -0.6961928009986877, -0.5250730514526367, -0.39491748809814453,
     -0.28444138169288635, -0.18477343022823334, -0.09105003625154495, 0.0,
     0.07958029955625534, 0.16093020141124725, 0.24611230194568634,
     0.33791524171829224, 0.44070982933044434, 0.5626170039176941,
     0.7229568362236023, 1.0], dtype=np.float32)
_MID = ((_LV[:-1] + _LV[1:]) * np.float32(0.5)).astype(np.float32)

_NC, _NS, _L = 2, 16, 16
_NW = _NC * _NS

_ROWS = 4096
_COLS = 4096
_SC_ROWS = 1024
_CHUNK_ROWS = 8
_SC_ROWS_PER_W = _SC_ROWS // _NW
_SC_CHUNKS = _SC_ROWS_PER_W // _CHUNK_ROWS
_BLOCKS_PER_CHUNK = _CHUNK_ROWS * _COLS // 64


_GROUPS_PER_CHUNK = _CHUNK_ROWS * _COLS // (16 * 64)


def _sc_body(x_hbm, o_hbm, buf_in, buf_out, mid_tbl, lv_tbl):
    c = lax.axis_index("c")
    s = lax.axis_index("s")
    wid = s * _NC + c
    row0 = wid * _SC_ROWS_PER_W
    lane = lax.iota(jnp.int32, _L)

    def chunk_body(k, carry):
        base = row0 + k * _CHUNK_ROWS
        pltpu.sync_copy(x_hbm.at[pl.ds(base, _CHUNK_ROWS)], buf_in)

        def group_body(g, carry2):
            row = jnp.broadcast_to(g // 4, (_L,)).astype(jnp.int32)
            col0 = (g % 4) * 1024 + lane * 64

            def mx(j, ams):
                a0, a1, a2, a3 = ams
                g0 = plsc.load_gather(buf_in, [row, col0 + 4 * j])
                g1 = plsc.load_gather(buf_in, [row, col0 + 4 * j + 1])
                g2 = plsc.load_gather(buf_in, [row, col0 + 4 * j + 2])
                g3 = plsc.load_gather(buf_in, [row, col0 + 4 * j + 3])
                return (jnp.maximum(a0, jnp.abs(g0)),
                        jnp.maximum(a1, jnp.abs(g1)),
                        jnp.maximum(a2, jnp.abs(g2)),
                        jnp.maximum(a3, jnp.abs(g3)))

            z = jnp.zeros((_L,), jnp.float32)
            a0, a1, a2, a3 = lax.fori_loop(0, 16, mx, (z, z, z, z))
            am = jnp.maximum(jnp.maximum(a0, a1), jnp.maximum(a2, a3))
            am = jnp.maximum(am, jnp.float32(1e-8))
            for i in range(15):
                plsc.store_scatter(
                    mid_tbl, [lane, jnp.broadcast_to(i, (_L,)).astype(jnp.int32)],
                    am * _MID[i])
            for i in range(16):
                plsc.store_scatter(
                    lv_tbl, [lane, jnp.broadcast_to(i, (_L,)).astype(jnp.int32)],
                    am * _LV[i])
            t7 = am * _MID[7]

            @plsc.parallel_loop(0, 64, unroll=2)
            def q_body(j):
                gv = plsc.load_gather(buf_in, [row, col0 + j])
                idx = jnp.where(gv > t7, 8, 0).astype(jnp.int32)
                for kk in (4, 2, 1):
                    t = plsc.load_gather(mid_tbl, [lane, idx + (kk - 1)])
                    idx = jnp.where(gv > t, idx + kk, idx)
                val = plsc.load_gather(lv_tbl, [lane, idx])
                plsc.store_scatter(buf_out, [row, col0 + j], val)

            return carry2

        lax.fori_loop(0, _GROUPS_PER_CHUNK, group_body, jnp.int32(0))
        pltpu.sync_copy(buf_out, o_hbm.at[pl.ds(base, _CHUNK_ROWS)])
        return carry

    lax.fori_loop(0, _SC_CHUNKS, chunk_body, jnp.int32(0))


_sc_nf4 = functools.partial(
    pl.kernel,
    out_type=jax.ShapeDtypeStruct((_SC_ROWS, _COLS), jnp.float32),
    mesh=plsc.VectorSubcoreMesh(core_axis_name="c", subcore_axis_name="s"),
    scratch_types=[
        pltpu.VMEM((_CHUNK_ROWS, _COLS), jnp.float32),
        pltpu.VMEM((_CHUNK_ROWS, _COLS), jnp.float32),
        pltpu.VMEM((_L, 16), jnp.float32),
        pltpu.VMEM((_L, 16), jnp.float32),
    ],
    compiler_params=pltpu.CompilerParams(needs_layout_passes=False),
)(_sc_body)


def _tc_kernel(x_ref, o_ref):
    cols = x_ref.shape[1]
    for k in range(cols // 128):
        c = x_ref[:, k * 128:(k + 1) * 128]
        a = jnp.abs(c)
        am_lo = jnp.max(a[:, :64], axis=1, keepdims=True)
        am_hi = jnp.max(a[:, 64:], axis=1, keepdims=True)
        am = jnp.concatenate(
            [jnp.broadcast_to(am_lo, (c.shape[0], 64)),
             jnp.broadcast_to(am_hi, (c.shape[0], 64))], axis=1)
        am = jnp.maximum(am, jnp.float32(1e-8))
        xn = c / am
        q = jnp.full(c.shape, _LV[0], dtype=jnp.float32)
        for i in range(15):
            q = jnp.where(xn > _MID[i], jnp.float32(_LV[i + 1]), q)
        o_ref[:, k * 128:(k + 1) * 128] = q * am


def kernel(x, levels):
    orig_shape = x.shape
    orig_dtype = x.dtype
    xf = x.astype(jnp.float32)
    rows, cols = xf.shape

    sc_out = _sc_nf4(xf)

    br = 256
    tc_rows = rows - _SC_ROWS
    off = _SC_ROWS // br
    tc_out = pl.pallas_call(
        _tc_kernel,
        grid=(tc_rows // br,),
        in_specs=[pl.BlockSpec((br, cols), lambda i: (i + off, 0))],
        out_specs=pl.BlockSpec((br, cols), lambda i: (i + off, 0)),
        out_shape=jax.ShapeDtypeStruct((rows, cols), jnp.float32),
    )(xf)

    out = lax.dynamic_update_slice(tc_out, sc_out, (0, 0))
    return out.reshape(orig_shape).astype(orig_dtype)


# speedup vs baseline: 2.6276x; 2.6276x over previous
import functools

import jax
import jax.numpy as jnp
import numpy as np
from jax import lax
from jax.experimental import pallas as pl
from jax.experimental.pallas import tpu as pltpu
from jax.experimental.pallas import tpu_sc as plsc

_LV = np.array(
    [-1.0, -0.6961928009986877, -0.5250730514526367, -0.39491748809814453,
     -0.28444138169288635, -0.18477343022823334, -0.09105003625154495, 0.0,
     0.07958029955625534, 0.16093020141124725, 0.24611230194568634,
     0.33791524171829224, 0.44070982933044434, 0.5626170039176941,
     0.7229568362236023, 1.0], dtype=np.float32)
_MID = ((_LV[:-1] + _LV[1:]) * np.float32(0.5)).astype(np.float32)

_NC, _NS, _L = 2, 16, 16
_NW = _NC * _NS

_ROWS = 4096
_COLS = 4096
_SC_ROWS = 1024
_CHUNK_ROWS = 4
_SC_ROWS_PER_W = _SC_ROWS // _NW
_SC_CHUNKS = _SC_ROWS_PER_W // _CHUNK_ROWS
_BLOCKS_PER_CHUNK = _CHUNK_ROWS * _COLS // 64


def _sc_compute(buf_in, buf_out):
    @plsc.parallel_loop(0, _BLOCKS_PER_CHUNK, unroll=4)
    def block_body(bi):
        r = bi >> 6
        cb = pl.multiple_of((bi & 63) << 6, 64)
        v = [buf_in[r, pl.ds(pl.multiple_of(cb + 16 * j, 16), 16)]
             for j in range(4)]
        a = jnp.maximum(jnp.maximum(jnp.abs(v[0]), jnp.abs(v[1])),
                        jnp.maximum(jnp.abs(v[2]), jnp.abs(v[3])))
        am = jnp.maximum(jnp.max(a), jnp.float32(1e-8))
        ts = [am * _MID[i] for i in range(15)]
        for j in range(4):
            q = jnp.full((_L,), _LV[0], jnp.float32)
            for i in range(15):
                q = jnp.where(v[j] > ts[i], jnp.float32(_LV[i + 1]), q)
            buf_out[r, pl.ds(pl.multiple_of(cb + 16 * j, 16), 16)] = q * am


def _sc_body(x_hbm, o_hbm, bi0, bi1, bo0, bo1, si0, si1, so0, so1):
    c = lax.axis_index("c")
    s = lax.axis_index("s")
    wid = s * _NC + c
    row0 = wid * _SC_ROWS_PER_W
    bufs_in, bufs_out = [bi0, bi1], [bo0, bo1]
    sems_in, sems_out = [si0, si1], [so0, so1]

    def in_copy(k):
        return pltpu.make_async_copy(
            x_hbm.at[pl.ds(row0 + k * _CHUNK_ROWS, _CHUNK_ROWS)],
            bufs_in[k % 2], sems_in[k % 2])

    def out_copy(k):
        return pltpu.make_async_copy(
            bufs_out[k % 2],
            o_hbm.at[pl.ds(row0 + k * _CHUNK_ROWS, _CHUNK_ROWS)],
            sems_out[k % 2])

    in_copy(0).start()
    for k in range(_SC_CHUNKS):
        if k + 1 < _SC_CHUNKS:
            in_copy(k + 1).start()
        in_copy(k).wait()
        if k >= 2:
            out_copy(k - 2).wait()
        _sc_compute(bufs_in[k % 2], bufs_out[k % 2])
        out_copy(k).start()
    for k in range(max(_SC_CHUNKS - 2, 0), _SC_CHUNKS):
        out_copy(k).wait()


_sc_nf4 = functools.partial(
    pl.kernel,
    out_type=jax.ShapeDtypeStruct((_SC_ROWS, _COLS), jnp.float32),
    mesh=plsc.VectorSubcoreMesh(core_axis_name="c", subcore_axis_name="s"),
    scratch_types=[
        pltpu.VMEM((_CHUNK_ROWS, _COLS), jnp.float32),
        pltpu.VMEM((_CHUNK_ROWS, _COLS), jnp.float32),
        pltpu.VMEM((_CHUNK_ROWS, _COLS), jnp.float32),
        pltpu.VMEM((_CHUNK_ROWS, _COLS), jnp.float32),
        pltpu.SemaphoreType.DMA,
        pltpu.SemaphoreType.DMA,
        pltpu.SemaphoreType.DMA,
        pltpu.SemaphoreType.DMA,
    ],
    compiler_params=pltpu.CompilerParams(needs_layout_passes=False),
)(_sc_body)


def _tc_kernel(x_ref, o_ref):
    cols = x_ref.shape[1]
    for k in range(cols // 128):
        c = x_ref[:, k * 128:(k + 1) * 128]
        a = jnp.abs(c)
        am_lo = jnp.max(a[:, :64], axis=1, keepdims=True)
        am_hi = jnp.max(a[:, 64:], axis=1, keepdims=True)
        am = jnp.concatenate(
            [jnp.broadcast_to(am_lo, (c.shape[0], 64)),
             jnp.broadcast_to(am_hi, (c.shape[0], 64))], axis=1)
        am = jnp.maximum(am, jnp.float32(1e-8))
        xn = c / am
        q = jnp.full(c.shape, _LV[0], dtype=jnp.float32)
        for i in range(15):
            q = jnp.where(xn > _MID[i], jnp.float32(_LV[i + 1]), q)
        o_ref[:, k * 128:(k + 1) * 128] = q * am


def kernel(x, levels):
    orig_shape = x.shape
    orig_dtype = x.dtype
    xf = x.astype(jnp.float32)
    rows, cols = xf.shape

    sc_out = _sc_nf4(xf)

    br = 256
    tc_rows = rows - _SC_ROWS
    off = _SC_ROWS // br
    tc_out = pl.pallas_call(
        _tc_kernel,
        grid=(tc_rows // br,),
        in_specs=[pl.BlockSpec((br, cols), lambda i: (i + off, 0))],
        out_specs=pl.BlockSpec((br, cols), lambda i: (i + off, 0)),
        out_shape=jax.ShapeDtypeStruct((rows, cols), jnp.float32),
    )(xf)

    out = lax.dynamic_update_slice(tc_out, sc_out, (0, 0))
    return out.reshape(orig_shape).astype(orig_dtype)


# speedup vs baseline: 2.6905x; 1.0239x over previous
import functools

import jax
import jax.numpy as jnp
import numpy as np
from jax import lax
from jax.experimental import pallas as pl
from jax.experimental.pallas import tpu as pltpu
from jax.experimental.pallas import tpu_sc as plsc

_LV = np.array(
    [-1.0, -0.6961928009986877, -0.5250730514526367, -0.39491748809814453,
     -0.28444138169288635, -0.18477343022823334, -0.09105003625154495, 0.0,
     0.07958029955625534, 0.16093020141124725, 0.24611230194568634,
     0.33791524171829224, 0.44070982933044434, 0.5626170039176941,
     0.7229568362236023, 1.0], dtype=np.float32)
_MID = ((_LV[:-1] + _LV[1:]) * np.float32(0.5)).astype(np.float32)

_NC, _NS, _L = 2, 16, 16
_NW = _NC * _NS

_ROWS = 4096
_COLS = 4096
_SC_ROWS = 1024
_CHUNK_ROWS = 4
_SC_ROWS_PER_W = _SC_ROWS // _NW
_SC_CHUNKS = _SC_ROWS_PER_W // _CHUNK_ROWS
_BLOCKS_PER_CHUNK = _CHUNK_ROWS * _COLS // 64


def _sc_compute(buf_in, buf_out):
    @plsc.parallel_loop(0, _BLOCKS_PER_CHUNK, unroll=4)
    def block_body(bi):
        r = bi >> 6
        cb = pl.multiple_of((bi & 63) << 6, 64)
        v = [buf_in[r, pl.ds(pl.multiple_of(cb + 16 * j, 16), 16)]
             for j in range(4)]
        a = jnp.maximum(jnp.maximum(jnp.abs(v[0]), jnp.abs(v[1])),
                        jnp.maximum(jnp.abs(v[2]), jnp.abs(v[3])))
        am = jnp.maximum(jnp.max(a), jnp.float32(1e-8))
        ts = [am * _MID[i] for i in range(15)]
        for j in range(4):
            q = jnp.full((_L,), _LV[0], jnp.float32)
            for i in range(15):
                q = jnp.where(v[j] > ts[i], jnp.float32(_LV[i + 1]), q)
            buf_out[r, pl.ds(pl.multiple_of(cb + 16 * j, 16), 16)] = q * am


def _sc_body(x_hbm, o_hbm, bi0, bi1, bo0, bo1, si0, si1, so0, so1):
    c = lax.axis_index("c")
    s = lax.axis_index("s")
    wid = s * _NC + c
    row0 = wid * _SC_ROWS_PER_W
    bufs_in, bufs_out = [bi0, bi1], [bo0, bo1]
    sems_in, sems_out = [si0, si1], [so0, so1]

    def in_copy(k):
        return pltpu.make_async_copy(
            x_hbm.at[pl.ds(row0 + k * _CHUNK_ROWS, _CHUNK_ROWS)],
            bufs_in[k % 2], sems_in[k % 2])

    def out_copy(k):
        return pltpu.make_async_copy(
            bufs_out[k % 2],
            o_hbm.at[pl.ds(row0 + k * _CHUNK_ROWS, _CHUNK_ROWS)],
            sems_out[k % 2])

    in_copy(0).start()
    for k in range(_SC_CHUNKS):
        if k + 1 < _SC_CHUNKS:
            in_copy(k + 1).start()
        in_copy(k).wait()
        if k >= 2:
            out_copy(k - 2).wait()
        _sc_compute(bufs_in[k % 2], bufs_out[k % 2])
        out_copy(k).start()
    for k in range(max(_SC_CHUNKS - 2, 0), _SC_CHUNKS):
        out_copy(k).wait()


_sc_nf4 = functools.partial(
    pl.kernel,
    out_type=jax.ShapeDtypeStruct((_SC_ROWS, _COLS), jnp.float32),
    mesh=plsc.VectorSubcoreMesh(core_axis_name="c", subcore_axis_name="s"),
    scratch_types=[
        pltpu.VMEM((_CHUNK_ROWS, _COLS), jnp.float32),
        pltpu.VMEM((_CHUNK_ROWS, _COLS), jnp.float32),
        pltpu.VMEM((_CHUNK_ROWS, _COLS), jnp.float32),
        pltpu.VMEM((_CHUNK_ROWS, _COLS), jnp.float32),
        pltpu.SemaphoreType.DMA,
        pltpu.SemaphoreType.DMA,
        pltpu.SemaphoreType.DMA,
        pltpu.SemaphoreType.DMA,
    ],
    compiler_params=pltpu.CompilerParams(needs_layout_passes=False),
)(_sc_body)


def _tc_kernel(x_ref, o_ref):
    cols = x_ref.shape[1]
    for k in range(cols // 128):
        c = x_ref[:, k * 128:(k + 1) * 128]
        a = jnp.abs(c)
        am_lo = jnp.max(a[:, :64], axis=1, keepdims=True)
        am_hi = jnp.max(a[:, 64:], axis=1, keepdims=True)
        am = jnp.concatenate(
            [jnp.broadcast_to(am_lo, (c.shape[0], 64)),
             jnp.broadcast_to(am_hi, (c.shape[0], 64))], axis=1)
        am = jnp.maximum(am, jnp.float32(1e-8))
        xn = c / am
        q = jnp.full(c.shape, _LV[0], dtype=jnp.float32)
        for i in range(15):
            q = jnp.where(xn > _MID[i], jnp.float32(_LV[i + 1]), q)
        o_ref[:, k * 128:(k + 1) * 128] = q * am


def _copy_kernel(src_ref, dst_ref, o_ref):
    del dst_ref
    o_ref[...] = src_ref[...]


def kernel(x, levels):
    orig_shape = x.shape
    orig_dtype = x.dtype
    xf = x.astype(jnp.float32)
    rows, cols = xf.shape

    sc_out = _sc_nf4(xf)

    br = 128
    tc_rows = rows - _SC_ROWS
    off = _SC_ROWS // br
    tc_out = pl.pallas_call(
        _tc_kernel,
        grid=(tc_rows // br,),
        in_specs=[pl.BlockSpec((br, cols), lambda i: (i + off, 0))],
        out_specs=pl.BlockSpec((br, cols), lambda i: (i + off, 0)),
        out_shape=jax.ShapeDtypeStruct((rows, cols), jnp.float32),
    )(xf)

    out = pl.pallas_call(
        _copy_kernel,
        grid=(_SC_ROWS // br,),
        in_specs=[pl.BlockSpec((br, cols), lambda i: (i, 0)),
                  pl.BlockSpec(memory_space=pl.ANY)],
        out_specs=pl.BlockSpec((br, cols), lambda i: (i, 0)),
        out_shape=jax.ShapeDtypeStruct((rows, cols), jnp.float32),
        input_output_aliases={1: 0},
    )(sc_out, tc_out)
    return out.reshape(orig_shape).astype(orig_dtype)
